# baseline (speedup 1.0000x reference)
import functools

import jax
import jax.numpy as jnp
from jax import lax
from jax.experimental import pallas as pl
from jax.experimental.pallas import tpu as pltpu
from jax.experimental.pallas import tpu_sc as plsc

N = 50000
K = 10
F = 128
NA = 12
NR = 23
LANES = 16

NW = 32
BPW = 1568
NPAD = NW * BPW
C = 16
CK = C * K
NCHUNK = BPW // C
NBUF = 7

NPAIR = 4

_sc_mesh = plsc.VectorSubcoreMesh(core_axis_name="c", subcore_axis_name="s")


@functools.partial(
    pl.kernel,
    mesh=_sc_mesh,
    compiler_params=pltpu.CompilerParams(use_tc_tiling_on_sc=False),
    out_type=jax.ShapeDtypeStruct((NPAIR * NPAD, LANES), jnp.float32),
    scratch_types=(
        [pltpu.VMEM((NCHUNK, CK), jnp.int32)]
        + [pltpu.VMEM((CK, LANES), jnp.float32)] * NBUF
        + [pltpu.VMEM((BPW, LANES), jnp.float32)]
        + [pltpu.SemaphoreType.DMA] * NBUF
    ),
)
def _sc_mean_gather(table, idx_all, out_hbm, idx_v, *scratch):
    wid = lax.axis_index("s") * 2 + lax.axis_index("c")
    bufs = scratch[:NBUF]
    out_v = scratch[NBUF]
    sems = scratch[NBUF + 1:]

    def do_pair(pair, _):
        pltpu.sync_copy(idx_all.at[pair * NW + wid], idx_v)
        for b in range(NBUF):
            pltpu.async_copy(table.at[idx_v.at[b]], bufs[b], sems[b])

        def body(j, _):
            for b in range(NBUF):
                ch = j * NBUF + b
                pltpu.make_async_copy(
                    table.at[idx_v.at[ch]], bufs[b], sems[b]).wait()
                for i in range(C):
                    s = bufs[b][i * K, :]
                    for k in range(1, K):
                        s = s + bufs[b][i * K + k, :]
                    out_v[ch * C + i, :] = s * (1.0 / K)
                nxt = ch + NBUF

                @pl.when(nxt < NCHUNK)
                def _fire():
                    pltpu.async_copy(
                        table.at[idx_v.at[nxt]], bufs[b], sems[b])
            return ()

        lax.fori_loop(0, NCHUNK // NBUF, body, ())
        pltpu.sync_copy(
            out_v, out_hbm.at[pl.ds(pair * NPAD + wid * BPW, BPW)])
        return ()

    lax.fori_loop(0, NPAIR, do_pair, ())


BT = 2048


def _tc_fused(a_ref, r_ref, s_ref, d_ref, wv_ref, wr_ref, wsr_ref, wdr_ref,
              o_ref):
    acc = jnp.dot(a_ref[...], wv_ref[...], preferred_element_type=jnp.float32)
    acc = acc + jnp.dot(r_ref[...], wr_ref[...],
                        preferred_element_type=jnp.float32)
    acc = acc + jnp.dot(s_ref[0], wsr_ref[...],
                        preferred_element_type=jnp.float32)
    acc = acc + jnp.dot(d_ref[0], wdr_ref[...],
                        preferred_element_type=jnp.float32)
    o_ref[...] = jnp.maximum(acc, 0.0)


def _make_tc_call(p):
    return pl.pallas_call(
        _tc_fused,
        grid=(pl.cdiv(N, BT),),
        in_specs=[
            pl.BlockSpec((BT, LANES), lambda i: (i, 0)),
            pl.BlockSpec((BT, 24), lambda i: (i, 0)),
            pl.BlockSpec((1, BT, LANES), lambda i: (2 * p, i, 0)),
            pl.BlockSpec((1, BT, LANES), lambda i: (2 * p + 1, i, 0)),
            pl.BlockSpec((LANES, F), lambda i: (0, 0)),
            pl.BlockSpec((24, F), lambda i: (0, 0)),
            pl.BlockSpec((LANES, F), lambda i: (0, 0)),
            pl.BlockSpec((LANES, F), lambda i: (0, 0)),
        ],
        out_specs=pl.BlockSpec((BT, F), lambda i: (i, 0)),
        out_shape=jax.ShapeDtypeStruct((N, F), jnp.float32),
    )


_tc_call0 = _make_tc_call(0)
_tc_call1 = _make_tc_call(1)


def _prep_idx(neigh, offset):
    flat = neigh.reshape(-1) + offset
    flat = jnp.pad(flat, (0, (NPAD - N) * K))
    return flat.reshape(NW, NCHUNK, CK)


def kernel(atoms0, residues0, same_neigh0, diff_neigh0,
           atoms1, residues1, same_neigh1, diff_neigh1,
           Wv, Wr, Wsr, Wdr):
    a0p = jnp.pad(atoms0, ((0, 0), (0, LANES - NA)))
    a1p = jnp.pad(atoms1, ((0, 0), (0, LANES - NA)))
    r0p = jnp.pad(residues0, ((0, 0), (0, 24 - NR)))
    r1p = jnp.pad(residues1, ((0, 0), (0, 24 - NR)))
    wv = jnp.pad(Wv, ((0, LANES - NA), (0, 0)))
    wr = jnp.pad(Wr, ((0, 24 - NR), (0, 0)))
    wsr = jnp.pad(Wsr, ((0, LANES - NA), (0, 0)))
    wdr = jnp.pad(Wdr, ((0, LANES - NA), (0, 0)))

    table = jnp.concatenate([a0p, a1p], axis=0)
    idx_all = jnp.concatenate([
        _prep_idx(same_neigh0, 0), _prep_idx(diff_neigh0, 0),
        _prep_idx(same_neigh1, N), _prep_idx(diff_neigh1, N)], axis=0)

    agg = _sc_mean_gather(table, idx_all).reshape(NPAIR, NPAD, LANES)

    out0 = _tc_call0(a0p, r0p, agg, agg, wv, wr, wsr, wdr)
    out1 = _tc_call1(a1p, r1p, agg, agg, wv, wr, wsr, wdr)
    return ((out0, same_neigh0, diff_neigh0), (out1, same_neigh1, diff_neigh1))


# speedup vs baseline: 1.0483x; 1.0483x over previous
import functools

import jax
import jax.numpy as jnp
from jax import lax
from jax.experimental import pallas as pl
from jax.experimental.pallas import tpu as pltpu
from jax.experimental.pallas import tpu_sc as plsc

N = 50000
K = 10
F = 128
NA = 12
NR = 23
LANES = 16

NW = 32
BPW = 1568
NPAD = NW * BPW
C = 8
CK = C * K
NCHUNK = BPW // C
NBUF = 7

NPAIR = 4

_sc_mesh = plsc.VectorSubcoreMesh(core_axis_name="c", subcore_axis_name="s")


@functools.partial(
    pl.kernel,
    mesh=_sc_mesh,
    compiler_params=pltpu.CompilerParams(use_tc_tiling_on_sc=False),
    out_type=jax.ShapeDtypeStruct((NPAIR * NPAD, LANES), jnp.float32),
    scratch_types=(
        [pltpu.VMEM((NCHUNK, CK), jnp.int32)]
        + [pltpu.VMEM((CK, LANES), jnp.float32)] * NBUF
        + [pltpu.VMEM((BPW, LANES), jnp.float32)]
        + [pltpu.SemaphoreType.DMA] * NBUF
    ),
)
def _sc_mean_gather(table, idx_all, out_hbm, idx_v, *scratch):
    wid = lax.axis_index("s") * 2 + lax.axis_index("c")
    bufs = scratch[:NBUF]
    out_v = scratch[NBUF]
    sems = scratch[NBUF + 1:]

    def do_pair(pair, _):
        pltpu.sync_copy(idx_all.at[pair * NW + wid], idx_v)
        for b in range(NBUF):
            pltpu.async_copy(table.at[idx_v.at[b]], bufs[b], sems[b])

        def body(j, _):
            for b in range(NBUF):
                ch = j * NBUF + b
                pltpu.make_async_copy(
                    table.at[idx_v.at[ch]], bufs[b], sems[b]).wait()
                for i in range(C):
                    s = bufs[b][i * K, :]
                    for k in range(1, K):
                        s = s + bufs[b][i * K + k, :]
                    out_v[ch * C + i, :] = s * (1.0 / K)
                nxt = ch + NBUF

                @pl.when(nxt < NCHUNK)
                def _fire():
                    pltpu.async_copy(
                        table.at[idx_v.at[nxt]], bufs[b], sems[b])
            return ()

        lax.fori_loop(0, NCHUNK // NBUF, body, ())
        pltpu.sync_copy(
            out_v, out_hbm.at[pl.ds(pair * NPAD + wid * BPW, BPW)])
        return ()

    lax.fori_loop(0, NPAIR, do_pair, ())


BT = 2048


def _tc_fused(a_ref, r_ref, s_ref, d_ref, wv_ref, wr_ref, wsr_ref, wdr_ref,
              o_ref):
    acc = jnp.dot(a_ref[...], wv_ref[...], preferred_element_type=jnp.float32)
    acc = acc + jnp.dot(r_ref[...], wr_ref[...],
                        preferred_element_type=jnp.float32)
    acc = acc + jnp.dot(s_ref[0], wsr_ref[...],
                        preferred_element_type=jnp.float32)
    acc = acc + jnp.dot(d_ref[0], wdr_ref[...],
                        preferred_element_type=jnp.float32)
    o_ref[...] = jnp.maximum(acc, 0.0)


def _make_tc_call(p):
    return pl.pallas_call(
        _tc_fused,
        grid=(pl.cdiv(N, BT),),
        in_specs=[
            pl.BlockSpec((BT, LANES), lambda i: (i, 0)),
            pl.BlockSpec((BT, 24), lambda i: (i, 0)),
            pl.BlockSpec((1, BT, LANES), lambda i: (2 * p, i, 0)),
            pl.BlockSpec((1, BT, LANES), lambda i: (2 * p + 1, i, 0)),
            pl.BlockSpec((LANES, F), lambda i: (0, 0)),
            pl.BlockSpec((24, F), lambda i: (0, 0)),
            pl.BlockSpec((LANES, F), lambda i: (0, 0)),
            pl.BlockSpec((LANES, F), lambda i: (0, 0)),
        ],
        out_specs=pl.BlockSpec((BT, F), lambda i: (i, 0)),
        out_shape=jax.ShapeDtypeStruct((N, F), jnp.float32),
    )


_tc_call0 = _make_tc_call(0)
_tc_call1 = _make_tc_call(1)


def _prep_idx(neigh, offset):
    flat = neigh.reshape(-1) + offset
    flat = jnp.pad(flat, (0, (NPAD - N) * K))
    return flat.reshape(NW, NCHUNK, CK)


def kernel(atoms0, residues0, same_neigh0, diff_neigh0,
           atoms1, residues1, same_neigh1, diff_neigh1,
           Wv, Wr, Wsr, Wdr):
    a0p = jnp.pad(atoms0, ((0, 0), (0, LANES - NA)))
    a1p = jnp.pad(atoms1, ((0, 0), (0, LANES - NA)))
    r0p = jnp.pad(residues0, ((0, 0), (0, 24 - NR)))
    r1p = jnp.pad(residues1, ((0, 0), (0, 24 - NR)))
    wv = jnp.pad(Wv, ((0, LANES - NA), (0, 0)))
    wr = jnp.pad(Wr, ((0, 24 - NR), (0, 0)))
    wsr = jnp.pad(Wsr, ((0, LANES - NA), (0, 0)))
    wdr = jnp.pad(Wdr, ((0, LANES - NA), (0, 0)))

    table = jnp.concatenate([a0p, a1p], axis=0)
    idx_all = jnp.concatenate([
        _prep_idx(same_neigh0, 0), _prep_idx(diff_neigh0, 0),
        _prep_idx(same_neigh1, N), _prep_idx(diff_neigh1, N)], axis=0)

    agg = _sc_mean_gather(table, idx_all).reshape(NPAIR, NPAD, LANES)

    out0 = _tc_call0(a0p, r0p, agg, agg, wv, wr, wsr, wdr)
    out1 = _tc_call1(a1p, r1p, agg, agg, wv, wr, wsr, wdr)
    return ((out0, same_neigh0, diff_neigh0), (out1, same_neigh1, diff_neigh1))


# speedup vs baseline: 1.3311x; 1.2699x over previous
import functools

import jax
import jax.numpy as jnp
from jax import lax
from jax.experimental import pallas as pl
from jax.experimental.pallas import tpu as pltpu
from jax.experimental.pallas import tpu_sc as plsc

N = 50000
K = 10
F = 128
NA = 12
NR = 23
LANES = 16

NW = 32
BPW = 1568
C = 8
CK = C * K
NCHUNK = BPW // C
NBUF = 7

_sc_mesh = plsc.VectorSubcoreMesh(core_axis_name="c", subcore_axis_name="s")


@functools.partial(
    pl.kernel,
    mesh=_sc_mesh,
    compiler_params=pltpu.CompilerParams(use_tc_tiling_on_sc=False),
    out_type=[jax.ShapeDtypeStruct((N, LANES), jnp.float32)] * 4,
    scratch_types=(
        [pltpu.VMEM((NCHUNK, CK), jnp.int32)]
        + [pltpu.VMEM((CK, LANES), jnp.float32)] * NBUF
        + [pltpu.VMEM((BPW, LANES), jnp.float32)]
        + [pltpu.SemaphoreType.DMA] * NBUF
    ),
)
def _sc_mean_gather(t0, i00, i01, t1, i10, i11,
                    o00, o01, o10, o11,
                    idx_v, *scratch):
    wid = lax.axis_index("s") * 2 + lax.axis_index("c")
    base = jnp.where(wid == NW - 1, N - BPW, wid * BPW)
    bufs = scratch[:NBUF]
    out_v = scratch[NBUF]
    sems = scratch[NBUF + 1:]

    def do_pair(idx_hbm, table, out_hbm):
        pltpu.sync_copy(idx_hbm.at[pl.ds(base // C, NCHUNK)], idx_v)
        for b in range(NBUF):
            pltpu.async_copy(table.at[idx_v.at[b]], bufs[b], sems[b])

        def body(j, _):
            for b in range(NBUF):
                ch = j * NBUF + b
                pltpu.make_async_copy(
                    table.at[idx_v.at[ch]], bufs[b], sems[b]).wait()
                for i in range(C):
                    s = bufs[b][i * K, :]
                    for k in range(1, K):
                        s = s + bufs[b][i * K + k, :]
                    out_v[ch * C + i, :] = s * (1.0 / K)
                nxt = ch + NBUF

                @pl.when(nxt < NCHUNK)
                def _fire():
                    pltpu.async_copy(
                        table.at[idx_v.at[nxt]], bufs[b], sems[b])
            return ()

        lax.fori_loop(0, NCHUNK // NBUF, body, ())
        pltpu.sync_copy(out_v, out_hbm.at[pl.ds(base, BPW)])

    do_pair(i00, t0, o00)
    do_pair(i01, t0, o01)
    do_pair(i10, t1, o10)
    do_pair(i11, t1, o11)


BT = 2048


def _tc_fused(a_ref, r_ref, s_ref, d_ref, wv_ref, wr_ref, wsr_ref, wdr_ref,
              o_ref):
    acc = jnp.dot(a_ref[...], wv_ref[...], preferred_element_type=jnp.float32)
    acc = acc + jnp.dot(r_ref[...], wr_ref[...],
                        preferred_element_type=jnp.float32)
    acc = acc + jnp.dot(s_ref[...], wsr_ref[...],
                        preferred_element_type=jnp.float32)
    acc = acc + jnp.dot(d_ref[...], wdr_ref[...],
                        preferred_element_type=jnp.float32)
    o_ref[...] = jnp.maximum(acc, 0.0)


_tc_call = pl.pallas_call(
    _tc_fused,
    grid=(pl.cdiv(N, BT),),
    in_specs=[
        pl.BlockSpec((BT, NA), lambda i: (i, 0)),
        pl.BlockSpec((BT, NR), lambda i: (i, 0)),
        pl.BlockSpec((BT, LANES), lambda i: (i, 0)),
        pl.BlockSpec((BT, LANES), lambda i: (i, 0)),
        pl.BlockSpec((NA, F), lambda i: (0, 0)),
        pl.BlockSpec((NR, F), lambda i: (0, 0)),
        pl.BlockSpec((LANES, F), lambda i: (0, 0)),
        pl.BlockSpec((LANES, F), lambda i: (0, 0)),
    ],
    out_specs=pl.BlockSpec((BT, F), lambda i: (i, 0)),
    out_shape=jax.ShapeDtypeStruct((N, F), jnp.float32),
)


def kernel(atoms0, residues0, same_neigh0, diff_neigh0,
           atoms1, residues1, same_neigh1, diff_neigh1,
           Wv, Wr, Wsr, Wdr):
    a0p = jnp.pad(atoms0, ((0, 0), (0, LANES - NA)))
    a1p = jnp.pad(atoms1, ((0, 0), (0, LANES - NA)))
    wsr = jnp.pad(Wsr, ((0, LANES - NA), (0, 0)))
    wdr = jnp.pad(Wdr, ((0, LANES - NA), (0, 0)))

    agg00, agg01, agg10, agg11 = _sc_mean_gather(
        a0p, same_neigh0.reshape(N // C, CK), diff_neigh0.reshape(N // C, CK),
        a1p, same_neigh1.reshape(N // C, CK), diff_neigh1.reshape(N // C, CK))

    out0 = _tc_call(atoms0, residues0, agg00, agg01, Wv, Wr, wsr, wdr)
    out1 = _tc_call(atoms1, residues1, agg10, agg11, Wv, Wr, wsr, wdr)
    return ((out0, same_neigh0, diff_neigh0), (out1, same_neigh1, diff_neigh1))


# speedup vs baseline: 1.4716x; 1.1055x over previous
import functools

import jax
import jax.numpy as jnp
from jax import lax
from jax.experimental import pallas as pl
from jax.experimental.pallas import tpu as pltpu
from jax.experimental.pallas import tpu_sc as plsc

N = 50000
K = 10
F = 128
NA = 12
NR = 23
LANES = 16

NW = 32
BPW = 1568
C = 8
CK = C * K
NCHUNK = BPW // C
NBUF = 7

_sc_mesh = plsc.VectorSubcoreMesh(core_axis_name="c", subcore_axis_name="s")


@functools.partial(
    pl.kernel,
    mesh=_sc_mesh,
    compiler_params=pltpu.CompilerParams(use_tc_tiling_on_sc=False),
    out_type=jax.ShapeDtypeStruct((N, 4 * LANES), jnp.float32),
    scratch_types=(
        [pltpu.VMEM((NCHUNK, CK), jnp.int32)]
        + [pltpu.VMEM((CK, LANES), jnp.float32)] * NBUF
        + [pltpu.VMEM((BPW, 4 * LANES), jnp.float32)]
        + [pltpu.SemaphoreType.DMA] * NBUF
    ),
)
def _sc_mean_gather(t0, i00, i01, t1, i10, i11, out_hbm, idx_v, *scratch):
    wid = lax.axis_index("s") * 2 + lax.axis_index("c")
    base = jnp.where(wid == NW - 1, N - BPW, wid * BPW)
    bufs = scratch[:NBUF]
    out_v = scratch[NBUF]
    sems = scratch[NBUF + 1:]

    def do_pair(p, idx_hbm, table):
        pltpu.sync_copy(idx_hbm.at[pl.ds(base // C, NCHUNK)], idx_v)
        for b in range(NBUF):
            pltpu.async_copy(table.at[idx_v.at[b]], bufs[b], sems[b])

        def body(j, _):
            for b in range(NBUF):
                ch = j * NBUF + b
                pltpu.make_async_copy(
                    table.at[idx_v.at[ch]], bufs[b], sems[b]).wait()
                for i in range(C):
                    s = bufs[b][i * K, :]
                    for k in range(1, K):
                        s = s + bufs[b][i * K + k, :]
                    out_v[ch * C + i, pl.ds(LANES * p, LANES)] = s * (1.0 / K)
                nxt = ch + NBUF

                @pl.when(nxt < NCHUNK)
                def _fire():
                    pltpu.async_copy(
                        table.at[idx_v.at[nxt]], bufs[b], sems[b])
            return ()

        lax.fori_loop(0, NCHUNK // NBUF, body, ())

    do_pair(0, i00, t0)
    do_pair(1, i01, t0)
    do_pair(2, i10, t1)
    do_pair(3, i11, t1)
    pltpu.sync_copy(out_v, out_hbm.at[pl.ds(base, BPW)])


BT = 2048


def _make_tc_fused(p):
    def _tc_fused(a_ref, r_ref, g_ref, wv_ref, wr_ref, wsr_ref, wdr_ref,
                  o_ref):
        acc = jnp.dot(a_ref[...], wv_ref[...],
                      preferred_element_type=jnp.float32)
        acc = acc + jnp.dot(r_ref[...], wr_ref[...],
                            preferred_element_type=jnp.float32)
        g = g_ref[...]
        acc = acc + jnp.dot(g[:, 2 * p * LANES:(2 * p + 1) * LANES],
                            wsr_ref[...], preferred_element_type=jnp.float32)
        acc = acc + jnp.dot(g[:, (2 * p + 1) * LANES:(2 * p + 2) * LANES],
                            wdr_ref[...], preferred_element_type=jnp.float32)
        o_ref[...] = jnp.maximum(acc, 0.0)
    return _tc_fused


def _make_tc_call(p):
    return pl.pallas_call(
        _make_tc_fused(p),
        grid=(pl.cdiv(N, BT),),
        in_specs=[
            pl.BlockSpec((BT, NA), lambda i: (i, 0)),
            pl.BlockSpec((BT, NR), lambda i: (i, 0)),
            pl.BlockSpec((BT, 4 * LANES), lambda i: (i, 0)),
            pl.BlockSpec((NA, F), lambda i: (0, 0)),
            pl.BlockSpec((NR, F), lambda i: (0, 0)),
            pl.BlockSpec((LANES, F), lambda i: (0, 0)),
            pl.BlockSpec((LANES, F), lambda i: (0, 0)),
        ],
        out_specs=pl.BlockSpec((BT, F), lambda i: (i, 0)),
        out_shape=jax.ShapeDtypeStruct((N, F), jnp.float32),
    )


_tc_call0 = _make_tc_call(0)
_tc_call1 = _make_tc_call(1)


def kernel(atoms0, residues0, same_neigh0, diff_neigh0,
           atoms1, residues1, same_neigh1, diff_neigh1,
           Wv, Wr, Wsr, Wdr):
    a0p = jnp.pad(atoms0, ((0, 0), (0, LANES - NA)))
    a1p = jnp.pad(atoms1, ((0, 0), (0, LANES - NA)))
    wsr = jnp.pad(Wsr, ((0, LANES - NA), (0, 0)))
    wdr = jnp.pad(Wdr, ((0, LANES - NA), (0, 0)))

    agg = _sc_mean_gather(
        a0p, same_neigh0.reshape(N // C, CK), diff_neigh0.reshape(N // C, CK),
        a1p, same_neigh1.reshape(N // C, CK), diff_neigh1.reshape(N // C, CK))

    out0 = _tc_call0(atoms0, residues0, agg, Wv, Wr, wsr, wdr)
    out1 = _tc_call1(atoms1, residues1, agg, Wv, Wr, wsr, wdr)
    return ((out0, same_neigh0, diff_neigh0), (out1, same_neigh1, diff_neigh1))


# speedup vs baseline: 1.6336x; 1.1101x over previous
import functools

import jax
import jax.numpy as jnp
from jax import lax
from jax.experimental import pallas as pl
from jax.experimental.pallas import tpu as pltpu
from jax.experimental.pallas import tpu_sc as plsc

N = 50000
K = 10
F = 128
NA = 12
NR = 23
LANES = 16

NW = 32
BPW = 1568
C = 8
CK = C * K
NCHUNK = BPW // C
NBUF = 7

_sc_mesh = plsc.VectorSubcoreMesh(core_axis_name="c", subcore_axis_name="s")


@functools.partial(
    pl.kernel,
    mesh=_sc_mesh,
    compiler_params=pltpu.CompilerParams(use_tc_tiling_on_sc=False),
    out_type=jax.ShapeDtypeStruct((N, 2 * LANES), jnp.float32),
    scratch_types=(
        [pltpu.VMEM((NCHUNK, CK), jnp.int32)]
        + [pltpu.VMEM((CK, LANES), jnp.float32)] * NBUF
        + [pltpu.VMEM((BPW, 2 * LANES), jnp.float32)]
        + [pltpu.SemaphoreType.DMA] * NBUF
    ),
)
def _sc_mean_gather(table, idx_s, idx_d, out_hbm, idx_v, *scratch):
    wid = lax.axis_index("s") * 2 + lax.axis_index("c")
    base = jnp.where(wid == NW - 1, N - BPW, wid * BPW)
    bufs = scratch[:NBUF]
    out_v = scratch[NBUF]
    sems = scratch[NBUF + 1:]

    def do_pair(p, idx_hbm, table):
        pltpu.sync_copy(idx_hbm.at[pl.ds(base // C, NCHUNK)], idx_v)
        for b in range(NBUF):
            pltpu.async_copy(table.at[idx_v.at[b]], bufs[b], sems[b])

        def body(j, _):
            for b in range(NBUF):
                ch = j * NBUF + b
                pltpu.make_async_copy(
                    table.at[idx_v.at[ch]], bufs[b], sems[b]).wait()
                for i in range(C):
                    s = bufs[b][i * K, :]
                    for k in range(1, K):
                        s = s + bufs[b][i * K + k, :]
                    out_v[ch * C + i, pl.ds(LANES * p, LANES)] = s * (1.0 / K)
                nxt = ch + NBUF

                @pl.when(nxt < NCHUNK)
                def _fire():
                    pltpu.async_copy(
                        table.at[idx_v.at[nxt]], bufs[b], sems[b])
            return ()

        lax.fori_loop(0, NCHUNK // NBUF, body, ())

    do_pair(0, idx_s, table)
    do_pair(1, idx_d, table)
    pltpu.sync_copy(out_v, out_hbm.at[pl.ds(base, BPW)])


BT = 2048


def _tc_fused(a_ref, r_ref, g_ref, wv_ref, wr_ref, wsr_ref, wdr_ref,
              o_ref):
    acc = jnp.dot(a_ref[...], wv_ref[...],
                  preferred_element_type=jnp.float32)
    acc = acc + jnp.dot(r_ref[...], wr_ref[...],
                        preferred_element_type=jnp.float32)
    g = g_ref[...]
    acc = acc + jnp.dot(g[:, :LANES], wsr_ref[...],
                        preferred_element_type=jnp.float32)
    acc = acc + jnp.dot(g[:, LANES:], wdr_ref[...],
                        preferred_element_type=jnp.float32)
    o_ref[...] = jnp.maximum(acc, 0.0)


_tc_call = pl.pallas_call(
    _tc_fused,
    grid=(pl.cdiv(N, BT),),
    in_specs=[
        pl.BlockSpec((BT, NA), lambda i: (i, 0)),
        pl.BlockSpec((BT, NR), lambda i: (i, 0)),
        pl.BlockSpec((BT, 2 * LANES), lambda i: (i, 0)),
        pl.BlockSpec((NA, F), lambda i: (0, 0)),
        pl.BlockSpec((NR, F), lambda i: (0, 0)),
        pl.BlockSpec((LANES, F), lambda i: (0, 0)),
        pl.BlockSpec((LANES, F), lambda i: (0, 0)),
    ],
    out_specs=pl.BlockSpec((BT, F), lambda i: (i, 0)),
    out_shape=jax.ShapeDtypeStruct((N, F), jnp.float32),
)


def kernel(atoms0, residues0, same_neigh0, diff_neigh0,
           atoms1, residues1, same_neigh1, diff_neigh1,
           Wv, Wr, Wsr, Wdr):
    a0p = jnp.pad(atoms0, ((0, 0), (0, LANES - NA)))
    a1p = jnp.pad(atoms1, ((0, 0), (0, LANES - NA)))
    wsr = jnp.pad(Wsr, ((0, LANES - NA), (0, 0)))
    wdr = jnp.pad(Wdr, ((0, LANES - NA), (0, 0)))

    agg0 = _sc_mean_gather(
        a0p, same_neigh0.reshape(N // C, CK), diff_neigh0.reshape(N // C, CK))
    agg1 = _sc_mean_gather(
        a1p, same_neigh1.reshape(N // C, CK), diff_neigh1.reshape(N // C, CK))

    out0 = _tc_call(atoms0, residues0, agg0, Wv, Wr, wsr, wdr)
    out1 = _tc_call(atoms1, residues1, agg1, Wv, Wr, wsr, wdr)
    return ((out0, same_neigh0, diff_neigh0), (out1, same_neigh1, diff_neigh1))


# speedup vs baseline: 1.6339x; 1.0002x over previous
import functools

import jax
import jax.numpy as jnp
from jax import lax
from jax.experimental import pallas as pl
from jax.experimental.pallas import tpu as pltpu
from jax.experimental.pallas import tpu_sc as plsc

N = 50000
K = 10
F = 128
NA = 12
NR = 23
LANES = 16

NW = 32
BPW = 1568
C = 8
CK = C * K
NCHUNK = BPW // C
NBUF = 7

_sc_mesh = plsc.VectorSubcoreMesh(core_axis_name="c", subcore_axis_name="s")


@functools.partial(
    pl.kernel,
    mesh=_sc_mesh,
    compiler_params=pltpu.CompilerParams(use_tc_tiling_on_sc=False),
    out_type=jax.ShapeDtypeStruct((N, 2 * LANES), jnp.float32),
    scratch_types=(
        [pltpu.VMEM((NCHUNK, CK), jnp.int32)]
        + [pltpu.VMEM((CK, LANES), jnp.float32)] * NBUF
        + [pltpu.VMEM((BPW, 2 * LANES), jnp.float32)]
        + [pltpu.SemaphoreType.DMA] * NBUF
    ),
)
def _sc_mean_gather(table, idx_s, idx_d, out_hbm, idx_v, *scratch):
    wid = lax.axis_index("s") * 2 + lax.axis_index("c")
    base = jnp.where(wid == NW - 1, N - BPW, wid * BPW)
    bufs = scratch[:NBUF]
    out_v = scratch[NBUF]
    sems = scratch[NBUF + 1:]

    def do_pair(p, idx_hbm, table):
        pltpu.sync_copy(idx_hbm.at[pl.ds(base // C, NCHUNK)], idx_v)
        for b in range(NBUF):
            pltpu.async_copy(table.at[idx_v.at[b]], bufs[b], sems[b])

        def body(j, _):
            for b in range(NBUF):
                ch = j * NBUF + b
                pltpu.make_async_copy(
                    table.at[idx_v.at[ch]], bufs[b], sems[b]).wait()
                for i in range(C):
                    s = bufs[b][i * K, :]
                    for k in range(1, K):
                        s = s + bufs[b][i * K + k, :]
                    out_v[ch * C + i, pl.ds(LANES * p, LANES)] = s * (1.0 / K)
                nxt = ch + NBUF

                @pl.when(nxt < NCHUNK)
                def _fire():
                    pltpu.async_copy(
                        table.at[idx_v.at[nxt]], bufs[b], sems[b])
            return ()

        lax.fori_loop(0, NCHUNK // NBUF, body, ())

    do_pair(0, idx_s, table)
    do_pair(1, idx_d, table)
    pltpu.sync_copy(out_v, out_hbm.at[pl.ds(base, BPW)])


BT = 2048


def _tc_fused(a_ref, r_ref, g_ref, wv_ref, wr_ref, wsr_ref, wdr_ref,
              o_ref):
    acc = jnp.dot(a_ref[...], wv_ref[...],
                  preferred_element_type=jnp.float32)
    acc = acc + jnp.dot(r_ref[...], wr_ref[...],
                        preferred_element_type=jnp.float32)
    g = g_ref[...]
    acc = acc + jnp.dot(g[:, :LANES], wsr_ref[...],
                        preferred_element_type=jnp.float32)
    acc = acc + jnp.dot(g[:, LANES:], wdr_ref[...],
                        preferred_element_type=jnp.float32)
    o_ref[...] = jnp.maximum(acc, 0.0)


_tc_call = pl.pallas_call(
    _tc_fused,
    grid=(pl.cdiv(N, BT),),
    in_specs=[
        pl.BlockSpec((BT, NA), lambda i: (i, 0)),
        pl.BlockSpec((BT, NR), lambda i: (i, 0)),
        pl.BlockSpec((BT, 2 * LANES), lambda i: (i, 0)),
        pl.BlockSpec((NA, F), lambda i: (0, 0)),
        pl.BlockSpec((NR, F), lambda i: (0, 0)),
        pl.BlockSpec((LANES, F), lambda i: (0, 0)),
        pl.BlockSpec((LANES, F), lambda i: (0, 0)),
    ],
    out_specs=pl.BlockSpec((BT, F), lambda i: (i, 0)),
    out_shape=jax.ShapeDtypeStruct((N, F), jnp.float32),
)


def kernel(atoms0, residues0, same_neigh0, diff_neigh0,
           atoms1, residues1, same_neigh1, diff_neigh1,
           Wv, Wr, Wsr, Wdr):
    a0p = jnp.pad(atoms0, ((0, 0), (0, LANES - NA)))
    a1p = jnp.pad(atoms1, ((0, 0), (0, LANES - NA)))
    wsr = jnp.pad(Wsr, ((0, LANES - NA), (0, 0)))
    wdr = jnp.pad(Wdr, ((0, LANES - NA), (0, 0)))

    agg0 = _sc_mean_gather(
        a0p, same_neigh0.reshape(N // C, CK), diff_neigh0.reshape(N // C, CK))
    out0 = _tc_call(atoms0, residues0, agg0, Wv, Wr, wsr, wdr)
    agg1 = _sc_mean_gather(
        a1p, same_neigh1.reshape(N // C, CK), diff_neigh1.reshape(N // C, CK))
    out1 = _tc_call(atoms1, residues1, agg1, Wv, Wr, wsr, wdr)
    return ((out0, same_neigh0, diff_neigh0), (out1, same_neigh1, diff_neigh1))
